# baseline (speedup 1.0000x reference)
import jax
import jax.numpy as jnp
from jax import lax
from jax.experimental import pallas as pl
from jax.experimental.pallas import tpu as pltpu

N = 65536
K = 256
M = 64
BLOCK_N = 1024
NBUF = 24
NSTEPS = N // BLOCK_N


def _mm_body(x_hbm, w_ref, b_ref, o_hbm, *rest):
    xbufs = rest[:NBUF]
    obufs = rest[NBUF : 2 * NBUF]
    insems, outsems = rest[2 * NBUF], rest[2 * NBUF + 1]

    def in_copy(i, s):
        return pltpu.make_async_copy(
            x_hbm.at[pl.ds(i * BLOCK_N, BLOCK_N), :], xbufs[s], insems.at[s]
        )

    def out_copy(i, s):
        return pltpu.make_async_copy(
            obufs[s], o_hbm.at[:, pl.ds(i * BLOCK_N, BLOCK_N)], outsems.at[s]
        )

    for i in range(NBUF):
        in_copy(i, i).start()
    for i in range(NSTEPS):
        s = i % NBUF
        in_copy(i, s).wait()
        if i >= NBUF:
            out_copy(i - NBUF, s).wait()
        obufs[s][...] = (
            lax.dot_general(
                w_ref[...],
                xbufs[s][...],
                (((1,), (1,)), ((), ())),
                preferred_element_type=jnp.float32,
            )
            + b_ref[...]
        )
        out_copy(i, s).start()
        if i + NBUF < NSTEPS:
            in_copy(i + NBUF, s).start()
    for i in range(NSTEPS - NBUF, NSTEPS):
        out_copy(i, i % NBUF).wait()


@jax.jit
def _matmul_t(x, w, bias_col):
    return pl.pallas_call(
        _mm_body,
        in_specs=[
            pl.BlockSpec(memory_space=pl.ANY),
            pl.BlockSpec(memory_space=pltpu.VMEM),
            pl.BlockSpec(memory_space=pltpu.VMEM),
        ],
        out_specs=pl.BlockSpec(memory_space=pl.ANY),
        out_shape=jax.ShapeDtypeStruct((M, N), jnp.float32),
        scratch_shapes=(
            [pltpu.VMEM((BLOCK_N, K), jnp.float32) for _ in range(NBUF)]
            + [pltpu.VMEM((M, BLOCK_N), jnp.float32) for _ in range(NBUF)]
            + [
                pltpu.SemaphoreType.DMA((NBUF,)),
                pltpu.SemaphoreType.DMA((NBUF,)),
            ]
        ),
    )(x, w, bias_col)


def kernel(input, weight, bias):
    out_t = _matmul_t(input, weight, bias.reshape(M, 1))
    return out_t.T


# speedup vs baseline: 1.0050x; 1.0050x over previous
import jax
import jax.numpy as jnp
from jax import lax
from jax.experimental import pallas as pl
from jax.experimental.pallas import tpu as pltpu

N = 65536
K = 256
M = 64
BLOCK_N = 2048
NBUF = 16
NSTEPS = N // BLOCK_N


def _mm_body(x_hbm, w_ref, b_ref, o_hbm, *rest):
    xbufs = rest[:NBUF]
    obufs = rest[NBUF : 2 * NBUF]
    insems, outsems = rest[2 * NBUF], rest[2 * NBUF + 1]

    def in_copy(i, s):
        return pltpu.make_async_copy(
            x_hbm.at[pl.ds(i * BLOCK_N, BLOCK_N), :], xbufs[s], insems.at[s]
        )

    def out_copy(i, s):
        return pltpu.make_async_copy(
            obufs[s], o_hbm.at[:, pl.ds(i * BLOCK_N, BLOCK_N)], outsems.at[s]
        )

    for i in range(NBUF):
        in_copy(i, i).start()
    for i in range(NSTEPS):
        s = i % NBUF
        in_copy(i, s).wait()
        if i >= NBUF:
            out_copy(i - NBUF, s).wait()
        obufs[s][...] = (
            lax.dot_general(
                w_ref[...],
                xbufs[s][...],
                (((1,), (1,)), ((), ())),
                preferred_element_type=jnp.float32,
            )
            + b_ref[...]
        )
        out_copy(i, s).start()
        if i + NBUF < NSTEPS:
            in_copy(i + NBUF, s).start()
    for i in range(NSTEPS - NBUF, NSTEPS):
        out_copy(i, i % NBUF).wait()


@jax.jit
def _matmul_t(x, w, bias_col):
    return pl.pallas_call(
        _mm_body,
        in_specs=[
            pl.BlockSpec(memory_space=pl.ANY),
            pl.BlockSpec(memory_space=pltpu.VMEM),
            pl.BlockSpec(memory_space=pltpu.VMEM),
        ],
        out_specs=pl.BlockSpec(memory_space=pl.ANY),
        out_shape=jax.ShapeDtypeStruct((M, N), jnp.float32),
        scratch_shapes=(
            [pltpu.VMEM((BLOCK_N, K), jnp.float32) for _ in range(NBUF)]
            + [pltpu.VMEM((M, BLOCK_N), jnp.float32) for _ in range(NBUF)]
            + [
                pltpu.SemaphoreType.DMA((NBUF,)),
                pltpu.SemaphoreType.DMA((NBUF,)),
            ]
        ),
    )(x, w, bias_col)


def kernel(input, weight, bias):
    out_t = _matmul_t(input, weight, bias.reshape(M, 1))
    return out_t.T
